# baseline (speedup 1.0000x reference)
import jax
import jax.numpy as jnp
from jax.experimental import pallas as pl
from jax.experimental.pallas import tpu as pltpu

_NUM_SEG = 256
_N = 100000
_D = 128
_BLK = 10000
_NBLK = _N // _BLK


def _body(x_ref, bt_ref, w1_ref, b1_ref, w2_ref, b2_ref, out_ref,
          s_acc, d_acc):
    i = pl.program_id(0)

    @pl.when(i == 0)
    def _init():
        s_acc[...] = jnp.zeros_like(s_acc)
        d_acc[...] = jnp.zeros_like(d_acc)

    x = x_ref[...]
    h = jnp.tanh(
        jnp.dot(x, w1_ref[...], preferred_element_type=jnp.float32)
        + b1_ref[...])
    s = (jnp.dot(h, w2_ref[...], preferred_element_type=jnp.float32)
         + b2_ref[...])
    ex = jnp.exp(s)

    bt = bt_ref[...]
    seg_ids = jax.lax.broadcasted_iota(jnp.int32, (_BLK, _NUM_SEG), 1)
    oh = (seg_ids == bt).astype(jnp.float32)

    xe = x * ex
    s_acc[...] += jax.lax.dot_general(
        oh, xe, (((0,), (0,)), ((), ())),
        preferred_element_type=jnp.float32)
    d_acc[...] += jnp.sum(oh * ex, axis=0, keepdims=True)

    @pl.when(i == _NBLK - 1)
    def _finish():
        inv = 1.0 / (d_acc[...] + 1e-16)
        r = jax.lax.broadcasted_iota(jnp.int32, (_NUM_SEG, _NUM_SEG), 0)
        c = jax.lax.broadcasted_iota(jnp.int32, (_NUM_SEG, _NUM_SEG), 1)
        diag_inv = jnp.where(r == c, inv, 0.0)
        out_ref[...] = jnp.dot(diag_inv, s_acc[...],
                               preferred_element_type=jnp.float32)


@jax.jit
def kernel(x, batch, W1, b1, W2, b2):
    bt2 = batch.astype(jnp.int32).reshape(_N, 1)
    b1r = b1.reshape(1, 64).astype(jnp.float32)
    b2r = b2.reshape(1, 1).astype(jnp.float32)
    out = pl.pallas_call(
        _body,
        grid=(_NBLK,),
        in_specs=[
            pl.BlockSpec((_BLK, _D), lambda i: (i, 0)),
            pl.BlockSpec((_BLK, 1), lambda i: (i, 0)),
            pl.BlockSpec((_D, 64), lambda i: (0, 0)),
            pl.BlockSpec((1, 64), lambda i: (0, 0)),
            pl.BlockSpec((64, 1), lambda i: (0, 0)),
            pl.BlockSpec((1, 1), lambda i: (0, 0)),
        ],
        out_specs=pl.BlockSpec((_NUM_SEG, _D), lambda i: (0, 0)),
        out_shape=jax.ShapeDtypeStruct((_NUM_SEG, _D), jnp.float32),
        scratch_shapes=[
            pltpu.VMEM((_NUM_SEG, _D), jnp.float32),
            pltpu.VMEM((1, _NUM_SEG), jnp.float32),
        ],
        compiler_params=pltpu.CompilerParams(
            dimension_semantics=("arbitrary",),
        ),
    )(x, bt2, W1, b1r, W2, b2r)
    return out
